# baseline (speedup 1.0000x reference)
import functools

import jax
import jax.numpy as jnp
from jax import lax
from jax.experimental import pallas as pl
from jax.experimental.pallas import tpu as pltpu
from jax.experimental.pallas import tpu_sc as plsc

M = 1000000
D = 64
C = 100
CPAD = 128
B = 16384

NC = 2
NS = 16
NW = NC * NS
RPW0 = 31248
RPW1 = 8

LISTCAP = B + 32
WTAB = RPW0 + RPW1 + 8
BLK = 2048
CB = 20000


def _blend_body(val_ref, cc_ref, out_ref):
    v = val_ref[...]
    cc = cc_ref[...]
    p = jnp.dot(v, cc.T, preferred_element_type=jnp.float32)
    v2 = jnp.sum(v * v, axis=1, keepdims=True)
    c2 = jnp.sum(cc * cc, axis=1)[None, :]
    lane = lax.broadcasted_iota(jnp.int32, (1, CPAD), 1)
    c2 = c2 + jnp.where(lane >= C, jnp.float32(1e30), jnp.float32(0.0))
    d = (v2 - 2.0 * p) + c2
    mn = jnp.min(d, axis=1, keepdims=True)
    lanes2d = lax.broadcasted_iota(jnp.int32, (BLK, CPAD), 1)
    cand = jnp.where(d == mn, lanes2d, CPAD)
    amin = jnp.min(cand, axis=1, keepdims=True)
    onehot = (lanes2d == amin).astype(jnp.float32)
    center = jnp.dot(onehot, cc, preferred_element_type=jnp.float32)
    out_ref[...] = 0.5 * v + 0.5 * center


def _blend(val, cc_pad):
    return pl.pallas_call(
        _blend_body,
        grid=(B // BLK,),
        in_specs=[
            pl.BlockSpec((BLK, D), lambda i: (i, 0)),
            pl.BlockSpec((CPAD, D), lambda i: (0, 0)),
        ],
        out_specs=pl.BlockSpec((BLK, D), lambda i: (i, 0)),
        out_shape=jax.ShapeDtypeStruct((B, D), jnp.float32),
    )(val, cc_pad)


def _copy_body(src_ref, dst_ref, sem):
    pltpu.async_copy(src_ref, dst_ref, sem).wait()


def _copy(mem):
    return pl.pallas_call(
        _copy_body,
        in_specs=[pl.BlockSpec(memory_space=pl.ANY)],
        out_specs=pl.BlockSpec(memory_space=pl.ANY),
        out_shape=jax.ShapeDtypeStruct((M, D), jnp.float32),
        scratch_shapes=[pltpu.SemaphoreType.DMA],
    )(mem)


def _sc_body(out_hbm, blended_hbm, idx_hbm,
             idx_v, rowl_v, slotl_v, wtab_v, dsem):
    cid = lax.axis_index("c")
    sid = lax.axis_index("s")
    wid = sid * NC + cid
    base = wid * RPW0 + 8 * jnp.minimum(wid, 8)
    size = jnp.where(wid < 8, RPW0 + RPW1, RPW0)

    pltpu.sync_copy(idx_hbm, idx_v)

    iota16 = lax.broadcasted_iota(jnp.int32, (16,), 0)

    def cbody(g, cnt):
        vec = idx_v[pl.ds(g * 16, 16)]
        m = (vec >= base) & (vec < base + size)
        pc = plsc.all_reduce_population_count(m)
        plsc.store_compressed(rowl_v.at[pl.ds(cnt, 16)], vec, mask=m)
        plsc.store_compressed(slotl_v.at[pl.ds(cnt, 16)], iota16 + g * 16,
                              mask=m)
        return cnt + pc[0]

    n = lax.fori_loop(0, B // 16, cbody, jnp.int32(0))

    lane0 = iota16 == 0

    def w1(i, _):
        sp = jnp.zeros((16,), jnp.int32) + i
        tg = plsc.load_gather(rowl_v, [sp])
        sl = plsc.load_gather(slotl_v, [sp])
        plsc.store_scatter(wtab_v, [tg - base], sl, mask=lane0)
        return 0

    lax.fori_loop(0, n, w1, 0)

    def w2(g, cnt):
        i0 = g * 16
        valid = (iota16 + i0) < n
        tg = rowl_v[pl.ds(i0, 16)]
        sl = slotl_v[pl.ds(i0, 16)]
        win = plsc.load_gather(wtab_v, [tg - base], mask=valid)
        m = valid & (win == sl)
        pc = plsc.all_reduce_population_count(m)
        plsc.store_compressed(rowl_v.at[pl.ds(cnt, 16)], tg, mask=m)
        plsc.store_compressed(slotl_v.at[pl.ds(cnt, 16)], sl, mask=m)
        return cnt + pc[0]

    n2 = lax.fori_loop(0, (n + 15) // 16, w2, jnp.int32(0))
    padto = ((n2 + 15) // 16) * 16

    @pl.when(n2 > 0)
    def _pad():
        lastpos = jnp.zeros((16,), jnp.int32) + (n2 - 1)
        rowl_v[pl.ds(n2, 16)] = plsc.load_gather(rowl_v, [lastpos])
        slotl_v[pl.ds(n2, 16)] = plsc.load_gather(slotl_v, [lastpos])

    def fire(i, _):
        i0 = i * 4
        for j in range(4):
            sp = jnp.zeros((16,), jnp.int32) + (i0 + j)
            tg = plsc.load_gather(rowl_v, [sp])[0]
            sl = plsc.load_gather(slotl_v, [sp])[0]
            pltpu.async_copy(blended_hbm.at[pl.ds(sl, 1)],
                             out_hbm.at[pl.ds(tg, 1)], dsem.at[j])
        return 0

    lax.fori_loop(0, padto // 4, fire, 0)

    def drain(i, _):
        for j in range(4):
            pltpu.make_async_copy(blended_hbm.at[pl.ds(0, 1)],
                                  out_hbm.at[pl.ds(base, 1)],
                                  dsem.at[j]).wait()
        return 0

    lax.fori_loop(0, padto // 4, drain, 0)


_sc_scatter = functools.partial(
    pl.kernel,
    mesh=plsc.VectorSubcoreMesh(core_axis_name="c", subcore_axis_name="s",
                                num_cores=NC, num_subcores=NS),
    compiler_params=pltpu.CompilerParams(needs_layout_passes=False),
    scratch_types=[
        pltpu.VMEM((B,), jnp.int32),
        pltpu.VMEM((LISTCAP,), jnp.int32),
        pltpu.VMEM((LISTCAP,), jnp.int32),
        pltpu.VMEM((WTAB,), jnp.int32),
        pltpu.SemaphoreType.DMA((4,)),
    ],
)(_sc_body)


def kernel(mem, val, cell_centers, idx):
    cc_pad = jnp.concatenate(
        [cell_centers, jnp.zeros((CPAD - C, D), jnp.float32)], axis=0)
    blended = _blend(val, cc_pad)
    out0 = _copy(mem)
    out_ref = jax.new_ref(out0)
    _sc_scatter(out_ref, blended, idx.astype(jnp.int32))
    return out_ref[...]


# speedup vs baseline: 12.5904x; 12.5904x over previous
import functools

import jax
import jax.numpy as jnp
from jax import lax
from jax.experimental import pallas as pl
from jax.experimental.pallas import tpu as pltpu
from jax.experimental.pallas import tpu_sc as plsc

M = 1000000
D = 64
C = 100
CPAD = 128
B = 16384

NC = 2
NS = 16
NW = NC * NS
RPW0 = 31248
RPW1 = 8

LISTCAP = B + 32
WTAB = RPW0 + RPW1 + 8
CB = 20000
NB = M // CB
VB = 336
VPAD = NB * VB


def _fused_body(src_ref, val_ref, cc_ref, dst_ref, bl_ref):
    dst_ref[...] = src_ref[...]
    v = val_ref[...]
    cc = cc_ref[...]
    p = jnp.dot(v, cc.T, preferred_element_type=jnp.float32)
    v2 = jnp.sum(v * v, axis=1, keepdims=True)
    c2 = jnp.sum(cc * cc, axis=1)[None, :]
    lane = lax.broadcasted_iota(jnp.int32, (1, CPAD), 1)
    c2 = c2 + jnp.where(lane >= C, jnp.float32(1e30), jnp.float32(0.0))
    d = (v2 - 2.0 * p) + c2
    mn = jnp.min(d, axis=1, keepdims=True)
    lanes2d = lax.broadcasted_iota(jnp.int32, (VB, CPAD), 1)
    cand = jnp.where(d == mn, lanes2d, CPAD)
    amin = jnp.min(cand, axis=1, keepdims=True)
    onehot = (lanes2d == amin).astype(jnp.float32)
    center = jnp.dot(onehot, cc, preferred_element_type=jnp.float32)
    bl_ref[...] = 0.5 * v + 0.5 * center


def _fused(mem, val_pad, cc_pad):
    return pl.pallas_call(
        _fused_body,
        grid=(NB,),
        in_specs=[
            pl.BlockSpec((CB, D), lambda i: (i, 0)),
            pl.BlockSpec((VB, D), lambda i: (i, 0)),
            pl.BlockSpec((CPAD, D), lambda i: (0, 0)),
        ],
        out_specs=[
            pl.BlockSpec((CB, D), lambda i: (i, 0)),
            pl.BlockSpec((VB, D), lambda i: (i, 0)),
        ],
        out_shape=[
            jax.ShapeDtypeStruct((M, D), jnp.float32),
            jax.ShapeDtypeStruct((VPAD, D), jnp.float32),
        ],
    )(mem, val_pad, cc_pad)


def _sc_body(out_hbm, blended_hbm, idx_hbm,
             idx_v, rowl_v, slotl_v, wtab_v, dsem):
    cid = lax.axis_index("c")
    sid = lax.axis_index("s")
    wid = sid * NC + cid
    base = wid * RPW0 + 8 * jnp.minimum(wid, 8)
    size = jnp.where(wid < 8, RPW0 + RPW1, RPW0)

    pltpu.sync_copy(idx_hbm, idx_v)

    iota16 = lax.broadcasted_iota(jnp.int32, (16,), 0)

    def cbody(g, cnt):
        vec = idx_v[pl.ds(g * 16, 16)]
        m = (vec >= base) & (vec < base + size)
        pc = plsc.all_reduce_population_count(m)
        plsc.store_compressed(rowl_v.at[pl.ds(cnt, 16)], vec, mask=m)
        plsc.store_compressed(slotl_v.at[pl.ds(cnt, 16)], iota16 + g * 16,
                              mask=m)
        return cnt + pc[0]

    n = lax.fori_loop(0, B // 16, cbody, jnp.int32(0))

    lane0 = iota16 == 0

    def w1(i, _):
        sp = jnp.zeros((16,), jnp.int32) + i
        tg = plsc.load_gather(rowl_v, [sp])
        sl = plsc.load_gather(slotl_v, [sp])
        plsc.store_scatter(wtab_v, [tg - base], sl, mask=lane0)
        return 0

    lax.fori_loop(0, n, w1, 0)

    def w2(g, cnt):
        i0 = g * 16
        valid = (iota16 + i0) < n
        tg = rowl_v[pl.ds(i0, 16)]
        sl = slotl_v[pl.ds(i0, 16)]
        win = plsc.load_gather(wtab_v, [tg - base], mask=valid)
        m = valid & (win == sl)
        pc = plsc.all_reduce_population_count(m)
        plsc.store_compressed(rowl_v.at[pl.ds(cnt, 16)], tg, mask=m)
        plsc.store_compressed(slotl_v.at[pl.ds(cnt, 16)], sl, mask=m)
        return cnt + pc[0]

    n2 = lax.fori_loop(0, (n + 15) // 16, w2, jnp.int32(0))
    padto = ((n2 + 15) // 16) * 16

    @pl.when(n2 > 0)
    def _pad():
        lastpos = jnp.zeros((16,), jnp.int32) + (n2 - 1)
        rowl_v[pl.ds(n2, 16)] = plsc.load_gather(rowl_v, [lastpos])
        slotl_v[pl.ds(n2, 16)] = plsc.load_gather(slotl_v, [lastpos])

    def fire(i, _):
        i0 = i * 4
        for j in range(4):
            sp = jnp.zeros((16,), jnp.int32) + (i0 + j)
            tg = plsc.load_gather(rowl_v, [sp])[0]
            sl = plsc.load_gather(slotl_v, [sp])[0]
            pltpu.async_copy(blended_hbm.at[pl.ds(sl, 1)],
                             out_hbm.at[pl.ds(tg, 1)], dsem.at[j])
        return 0

    lax.fori_loop(0, padto // 4, fire, 0)

    def drain(i, _):
        for j in range(4):
            pltpu.make_async_copy(blended_hbm.at[pl.ds(0, 1)],
                                  out_hbm.at[pl.ds(base, 1)],
                                  dsem.at[j]).wait()
        return 0

    lax.fori_loop(0, padto // 4, drain, 0)


_sc_scatter = functools.partial(
    pl.kernel,
    mesh=plsc.VectorSubcoreMesh(core_axis_name="c", subcore_axis_name="s",
                                num_cores=NC, num_subcores=NS),
    compiler_params=pltpu.CompilerParams(needs_layout_passes=False),
    scratch_types=[
        pltpu.VMEM((B,), jnp.int32),
        pltpu.VMEM((LISTCAP,), jnp.int32),
        pltpu.VMEM((LISTCAP,), jnp.int32),
        pltpu.VMEM((WTAB,), jnp.int32),
        pltpu.SemaphoreType.DMA((4,)),
    ],
)(_sc_body)


def kernel(mem, val, cell_centers, idx):
    cc_pad = jnp.concatenate(
        [cell_centers, jnp.zeros((CPAD - C, D), jnp.float32)], axis=0)
    val_pad = jnp.concatenate(
        [val, jnp.zeros((VPAD - B, D), jnp.float32)], axis=0)
    out0, blended = _fused(mem, val_pad, cc_pad)
    out_ref = jax.new_ref(out0)
    _sc_scatter(out_ref, blended, idx.astype(jnp.int32))
    return out_ref[...]


# speedup vs baseline: 12.6165x; 1.0021x over previous
import functools

import jax
import jax.numpy as jnp
from jax import lax
from jax.experimental import pallas as pl
from jax.experimental.pallas import tpu as pltpu
from jax.experimental.pallas import tpu_sc as plsc

M = 1000000
D = 64
C = 100
CPAD = 128
B = 16384

NC = 2
NS = 16
NW = NC * NS
RPW0 = 31248
RPW1 = 8

LISTCAP = B + 32
WTAB = RPW0 + RPW1 + 8
CB = 25000
NB = M // CB
VB = 416
VPAD = NB * VB


def _fused_body(src_ref, val_ref, cc_ref, dst_ref, bl_ref):
    dst_ref[...] = src_ref[...]
    v = val_ref[...]
    cc = cc_ref[...]
    p = jnp.dot(v, cc.T, preferred_element_type=jnp.float32)
    v2 = jnp.sum(v * v, axis=1, keepdims=True)
    c2 = jnp.sum(cc * cc, axis=1)[None, :]
    lane = lax.broadcasted_iota(jnp.int32, (1, CPAD), 1)
    c2 = c2 + jnp.where(lane >= C, jnp.float32(1e30), jnp.float32(0.0))
    d = (v2 - 2.0 * p) + c2
    mn = jnp.min(d, axis=1, keepdims=True)
    lanes2d = lax.broadcasted_iota(jnp.int32, (VB, CPAD), 1)
    cand = jnp.where(d == mn, lanes2d, CPAD)
    amin = jnp.min(cand, axis=1, keepdims=True)
    onehot = (lanes2d == amin).astype(jnp.float32)
    center = jnp.dot(onehot, cc, preferred_element_type=jnp.float32)
    bl_ref[...] = 0.5 * v + 0.5 * center


def _fused(mem, val_pad, cc_pad):
    return pl.pallas_call(
        _fused_body,
        grid=(NB,),
        in_specs=[
            pl.BlockSpec((CB, D), lambda i: (i, 0)),
            pl.BlockSpec((VB, D), lambda i: (i, 0)),
            pl.BlockSpec((CPAD, D), lambda i: (0, 0)),
        ],
        out_specs=[
            pl.BlockSpec((CB, D), lambda i: (i, 0)),
            pl.BlockSpec((VB, D), lambda i: (i, 0)),
        ],
        out_shape=[
            jax.ShapeDtypeStruct((M, D), jnp.float32),
            jax.ShapeDtypeStruct((VPAD, D), jnp.float32),
        ],
    )(mem, val_pad, cc_pad)


def _sc_body(out_hbm, blended_hbm, idx_hbm,
             idx_v, rowl_v, slotl_v, wtab_v, dsem):
    cid = lax.axis_index("c")
    sid = lax.axis_index("s")
    wid = sid * NC + cid
    base = wid * RPW0 + 8 * jnp.minimum(wid, 8)
    size = jnp.where(wid < 8, RPW0 + RPW1, RPW0)

    pltpu.sync_copy(idx_hbm, idx_v)

    iota16 = lax.broadcasted_iota(jnp.int32, (16,), 0)

    def cbody(g, cnt):
        vec = idx_v[pl.ds(g * 16, 16)]
        m = (vec >= base) & (vec < base + size)
        pc = plsc.all_reduce_population_count(m)
        plsc.store_compressed(rowl_v.at[pl.ds(cnt, 16)], vec, mask=m)
        plsc.store_compressed(slotl_v.at[pl.ds(cnt, 16)], iota16 + g * 16,
                              mask=m)
        return cnt + pc[0]

    n = lax.fori_loop(0, B // 16, cbody, jnp.int32(0))

    lane0 = iota16 == 0

    def w1(i, _):
        sp = jnp.zeros((16,), jnp.int32) + i
        tg = plsc.load_gather(rowl_v, [sp])
        sl = plsc.load_gather(slotl_v, [sp])
        plsc.store_scatter(wtab_v, [tg - base], sl, mask=lane0)
        return 0

    lax.fori_loop(0, n, w1, 0)

    def w2(g, cnt):
        i0 = g * 16
        valid = (iota16 + i0) < n
        tg = rowl_v[pl.ds(i0, 16)]
        sl = slotl_v[pl.ds(i0, 16)]
        win = plsc.load_gather(wtab_v, [tg - base], mask=valid)
        m = valid & (win == sl)
        pc = plsc.all_reduce_population_count(m)
        plsc.store_compressed(rowl_v.at[pl.ds(cnt, 16)], tg, mask=m)
        plsc.store_compressed(slotl_v.at[pl.ds(cnt, 16)], sl, mask=m)
        return cnt + pc[0]

    n2 = lax.fori_loop(0, (n + 15) // 16, w2, jnp.int32(0))
    padto = ((n2 + 15) // 16) * 16

    @pl.when(n2 > 0)
    def _pad():
        lastpos = jnp.zeros((16,), jnp.int32) + (n2 - 1)
        rowl_v[pl.ds(n2, 16)] = plsc.load_gather(rowl_v, [lastpos])
        slotl_v[pl.ds(n2, 16)] = plsc.load_gather(slotl_v, [lastpos])

    def fire(i, _):
        i0 = i * 4
        for j in range(4):
            sp = jnp.zeros((16,), jnp.int32) + (i0 + j)
            tg = plsc.load_gather(rowl_v, [sp])[0]
            sl = plsc.load_gather(slotl_v, [sp])[0]
            pltpu.async_copy(blended_hbm.at[pl.ds(sl, 1)],
                             out_hbm.at[pl.ds(tg, 1)], dsem.at[j])
        return 0

    lax.fori_loop(0, padto // 4, fire, 0)

    def drain(i, _):
        for j in range(4):
            pltpu.make_async_copy(blended_hbm.at[pl.ds(0, 1)],
                                  out_hbm.at[pl.ds(base, 1)],
                                  dsem.at[j]).wait()
        return 0

    lax.fori_loop(0, padto // 4, drain, 0)


_sc_scatter = functools.partial(
    pl.kernel,
    mesh=plsc.VectorSubcoreMesh(core_axis_name="c", subcore_axis_name="s",
                                num_cores=NC, num_subcores=NS),
    compiler_params=pltpu.CompilerParams(needs_layout_passes=False),
    scratch_types=[
        pltpu.VMEM((B,), jnp.int32),
        pltpu.VMEM((LISTCAP,), jnp.int32),
        pltpu.VMEM((LISTCAP,), jnp.int32),
        pltpu.VMEM((WTAB,), jnp.int32),
        pltpu.SemaphoreType.DMA((4,)),
    ],
)(_sc_body)


def kernel(mem, val, cell_centers, idx):
    cc_pad = jnp.concatenate(
        [cell_centers, jnp.zeros((CPAD - C, D), jnp.float32)], axis=0)
    val_pad = jnp.concatenate(
        [val, jnp.zeros((VPAD - B, D), jnp.float32)], axis=0)
    out0, blended = _fused(mem, val_pad, cc_pad)
    out_ref = jax.new_ref(out0)
    _sc_scatter(out_ref, blended, idx.astype(jnp.int32))
    return out_ref[...]
